# baseline (speedup 1.0000x reference)
import jax
import jax.numpy as jnp
from jax import lax
from jax.experimental import pallas as pl
from jax.experimental.pallas import tpu as pltpu
from jax.experimental.pallas import tpu_sc as plsc

BATCH = 16384
NUM_FEATURES = 5
EMBED_DIM = 32
OUT_DIM = NUM_FEATURES + EMBED_DIM

NC = 2
NS = 16
LANES = 16
NW = NC * NS
BPW = BATCH // NW


def _sc_body(idx_hbm, table_hbm, emb_hbm, idx_v, rows, sem):
    wid = lax.axis_index("s") * NC + lax.axis_index("c")
    base = wid * BPW

    pltpu.sync_copy(idx_hbm.at[pl.ds(base, BPW)], idx_v)

    pltpu.async_copy(table_hbm.at[idx_v], rows, sem).wait()

    pltpu.sync_copy(rows, emb_hbm.at[pl.ds(base, BPW)])


@jax.jit
def kernel(x, table):
    idx = x[:, NUM_FEATURES].astype(jnp.int32)
    mesh = plsc.VectorSubcoreMesh(core_axis_name="c", subcore_axis_name="s")
    run = pl.kernel(
        _sc_body,
        out_type=jax.ShapeDtypeStruct((BATCH, EMBED_DIM), jnp.float32),
        mesh=mesh,
        compiler_params=pltpu.CompilerParams(use_tc_tiling_on_sc=False),
        scratch_types=[
            pltpu.VMEM((BPW,), jnp.int32),
            pltpu.VMEM((BPW, EMBED_DIM), jnp.float32),
            pltpu.SemaphoreType.DMA,
        ],
    )
    emb = run(idx, table)
    return jnp.concatenate([x[:, :NUM_FEATURES], emb], axis=1)
